# baseline (speedup 1.0000x reference)
import functools
import math

import jax
import jax.numpy as jnp
from jax import lax
from jax.experimental import pallas as pl
from jax.experimental.pallas import tpu as pltpu
from jax.experimental.pallas import tpu_sc as plsc

_SMOOTHING = 0.1
_CONFIDENCE = 1.0 - _SMOOTHING
_N = 1024
_V = 100000
_EPS = _SMOOTHING / (_V - 1)
_CONST = _N * ((_V - 1) * _EPS * math.log(_EPS) + _CONFIDENCE * math.log(_CONFIDENCE))

_NW = 32
_L = 16
_PER = _N // _NW

_RB = 8
_NB = _N // _RB


def _sc_body(x_hbm, tgt_hbm, out_hbm, gbuf, tv, accv, gsem):
    wid = lax.axis_index("s") * 2 + lax.axis_index("c")
    base = wid * _PER

    pltpu.sync_copy(tgt_hbm.at[pl.ds(base, _PER)], tv)
    tva = tv[pl.ds(0, _L)]
    tvb = tv[pl.ds(_L, _L)]

    def scalar_t(k):
        return (tva if k < _L else tvb)[k % _L]

    handles = []
    for k in range(_PER):
        t = pl.multiple_of(scalar_t(k) & (-128), 128)
        row8 = base + 8 * (k // 8)
        handles.append(
            pltpu.async_copy(
                x_hbm.at[pl.ds(row8, 8), pl.ds(t, 128)],
                gbuf.at[pl.ds(8 * k, 8), :],
                gsem,
            )
        )
    for h in handles:
        h.wait()

    iota = lax.iota(jnp.int32, _L)
    gacc = jnp.zeros((_L,), jnp.float32)
    for k in range(_PER):
        tmod = scalar_t(k) & 127
        row = 8 * k + (k % 8)
        for j in range(8):
            v = gbuf[row, pl.ds(_L * j, _L)]
            gacc = gacc + jnp.where(iota + _L * j == tmod, v, 0.0)
    accv[...] = gacc
    pltpu.sync_copy(accv, out_hbm.at[wid])


_sc_call = functools.partial(
    pl.kernel,
    mesh=plsc.VectorSubcoreMesh(core_axis_name="c", subcore_axis_name="s"),
    out_type=jax.ShapeDtypeStruct((_NW, _L), jnp.float32),
    scratch_types=[
        pltpu.VMEM((8 * _PER, 128), jnp.float32),
        pltpu.VMEM((_PER,), jnp.int32),
        pltpu.VMEM((_L,), jnp.float32),
        pltpu.SemaphoreType.DMA,
    ],
)(_sc_body)


def _tc_body(x_ref, out_ref):
    b = pl.program_id(0)

    @pl.when(b == 0)
    def _init():
        out_ref[...] = jnp.zeros((1, 1), jnp.float32)

    out_ref[...] += jnp.sum(x_ref[...]).reshape(1, 1)


def _combine_body(g_ref, s_ref, out_ref):
    gsum = jnp.sum(g_ref[...])
    out_ref[...] = (
        jnp.float32(_CONST)
        - jnp.float32(_EPS) * s_ref[0, 0]
        - jnp.float32(_CONFIDENCE - _EPS) * gsum
    ).reshape(1, 1)


def kernel(x, target):
    tgt = target.astype(jnp.int32)
    parts = _sc_call(x, tgt)
    s = pl.pallas_call(
        _tc_body,
        grid=(_NB,),
        in_specs=[pl.BlockSpec((_RB, _V), lambda b: (b, 0))],
        out_specs=pl.BlockSpec((1, 1), lambda b: (0, 0)),
        out_shape=jax.ShapeDtypeStruct((1, 1), jnp.float32),
        compiler_params=pltpu.CompilerParams(
            dimension_semantics=("arbitrary",),
        ),
    )(x)
    out = pl.pallas_call(
        _combine_body,
        in_specs=[
            pl.BlockSpec((_NW, _L), lambda: (0, 0)),
            pl.BlockSpec((1, 1), lambda: (0, 0)),
        ],
        out_specs=pl.BlockSpec((1, 1), lambda: (0, 0)),
        out_shape=jax.ShapeDtypeStruct((1, 1), jnp.float32),
    )(parts, s)
    return out[0, 0]


# speedup vs baseline: 1.1191x; 1.1191x over previous
import functools
import math

import jax
import jax.numpy as jnp
from jax import lax
from jax.experimental import pallas as pl
from jax.experimental.pallas import tpu as pltpu
from jax.experimental.pallas import tpu_sc as plsc

_SMOOTHING = 0.1
_CONFIDENCE = 1.0 - _SMOOTHING
_N = 1024
_V = 100000
_EPS = _SMOOTHING / (_V - 1)
_CONST = _N * ((_V - 1) * _EPS * math.log(_EPS) + _CONFIDENCE * math.log(_CONFIDENCE))

_NW = 32
_L = 16
_PER = _N // _NW

_RB = 64
_NB = _N // _RB


def _sc_body(x_hbm, tgt_hbm, out_hbm, gbuf, tv, accv, gsem):
    wid = lax.axis_index("s") * 2 + lax.axis_index("c")
    base = wid * _PER

    pltpu.sync_copy(tgt_hbm.at[pl.ds(base, _PER)], tv)
    tva = tv[pl.ds(0, _L)]
    tvb = tv[pl.ds(_L, _L)]

    def scalar_t(k):
        return (tva if k < _L else tvb)[k % _L]

    handles = []
    for k in range(_PER):
        t = pl.multiple_of(scalar_t(k) & (-128), 128)
        row8 = base + 8 * (k // 8)
        handles.append(
            pltpu.async_copy(
                x_hbm.at[pl.ds(row8, 8), pl.ds(t, 128)],
                gbuf.at[pl.ds(8 * k, 8), :],
                gsem,
            )
        )
    for h in handles:
        h.wait()

    iota = lax.iota(jnp.int32, _L)
    gacc = jnp.zeros((_L,), jnp.float32)
    for k in range(_PER):
        tmod = scalar_t(k) & 127
        row = 8 * k + (k % 8)
        for j in range(8):
            v = gbuf[row, pl.ds(_L * j, _L)]
            gacc = gacc + jnp.where(iota + _L * j == tmod, v, 0.0)
    accv[...] = gacc
    pltpu.sync_copy(accv, out_hbm.at[wid])


_sc_call = functools.partial(
    pl.kernel,
    mesh=plsc.VectorSubcoreMesh(core_axis_name="c", subcore_axis_name="s"),
    out_type=jax.ShapeDtypeStruct((_NW, _L), jnp.float32),
    scratch_types=[
        pltpu.VMEM((8 * _PER, 128), jnp.float32),
        pltpu.VMEM((_PER,), jnp.int32),
        pltpu.VMEM((_L,), jnp.float32),
        pltpu.SemaphoreType.DMA,
    ],
)(_sc_body)


def _tc_body(x_ref, out_ref):
    b = pl.program_id(0)

    @pl.when(b == 0)
    def _init():
        out_ref[...] = jnp.zeros((1, 1), jnp.float32)

    out_ref[...] += jnp.sum(x_ref[...]).reshape(1, 1)


def _combine_body(g_ref, s_ref, out_ref):
    gsum = jnp.sum(g_ref[...])
    out_ref[...] = (
        jnp.float32(_CONST)
        - jnp.float32(_EPS) * s_ref[0, 0]
        - jnp.float32(_CONFIDENCE - _EPS) * gsum
    ).reshape(1, 1)


def kernel(x, target):
    tgt = target.astype(jnp.int32)
    parts = _sc_call(x, tgt)
    s = pl.pallas_call(
        _tc_body,
        grid=(_NB,),
        in_specs=[pl.BlockSpec((_RB, _V), lambda b: (b, 0))],
        out_specs=pl.BlockSpec((1, 1), lambda b: (0, 0)),
        out_shape=jax.ShapeDtypeStruct((1, 1), jnp.float32),
        compiler_params=pltpu.CompilerParams(
            dimension_semantics=("arbitrary",),
        ),
    )(x)
    out = pl.pallas_call(
        _combine_body,
        in_specs=[
            pl.BlockSpec((_NW, _L), lambda: (0, 0)),
            pl.BlockSpec((1, 1), lambda: (0, 0)),
        ],
        out_specs=pl.BlockSpec((1, 1), lambda: (0, 0)),
        out_shape=jax.ShapeDtypeStruct((1, 1), jnp.float32),
    )(parts, s)
    return out[0, 0]
